# baseline (speedup 1.0000x reference)
import jax
import jax.numpy as jnp
from jax.experimental import pallas as pl
from jax.experimental.pallas import tpu as pltpu


def _moe_body(F, x_ref, scale_ref, gate_w_ref, gate_b_ref,
              w1_ref, b1_ref, w2_ref, b2_ref,
              out_ref, t_scr, w_scr):
    e = pl.program_id(0)

    @pl.when(e == 0)
    def _routing():
        x = x_ref[...]
        t = x * jax.lax.rsqrt(jnp.mean(x * x, axis=-1, keepdims=True) + 1e-5)
        t = t * scale_ref[...]
        t_scr[...] = t
        g = jax.lax.dot_general(
            t, gate_w_ref[...], (((1,), (1,)), ((), ())),
            preferred_element_type=jnp.float32) + gate_b_ref[...]
        ncols = g.shape[-1]
        col = jax.lax.broadcasted_iota(jnp.int32, g.shape, 1)
        v1 = jnp.max(g, axis=-1, keepdims=True)
        e1 = jnp.min(jnp.where(g == v1, col, ncols), axis=-1, keepdims=True)
        first1 = (col == e1)
        g2 = jnp.where(first1, -1e30, g)
        v2 = jnp.max(g2, axis=-1, keepdims=True)
        e2 = jnp.min(jnp.where(g2 == v2, col, ncols), axis=-1, keepdims=True)
        first2 = (col == e2)
        p1 = jax.nn.sigmoid(v1 - v2)
        p2 = 1.0 - p1
        w_scr[...] = jnp.where(first1, p1, 0.0) + jnp.where(first2, p2, 0.0)
        out_ref[...] = x

    t = t_scr[...]
    h = jax.lax.dot_general(
        t.astype(jnp.bfloat16), w1_ref[0].astype(jnp.bfloat16),
        (((1,), (1,)), ((), ())),
        preferred_element_type=jnp.float32) + b1_ref[0]
    x_glu = h[:, :F]
    x_lin = h[:, F:]
    a = x_glu * jax.nn.sigmoid(1.702 * x_glu) * (x_lin + 1.0)
    o = jax.lax.dot_general(
        a.astype(jnp.bfloat16), w2_ref[0].astype(jnp.bfloat16),
        (((1,), (1,)), ((), ())),
        preferred_element_type=jnp.float32) + b2_ref[0]
    w_all = w_scr[...]
    ecol = jax.lax.broadcasted_iota(jnp.int32, w_all.shape, 1)
    wcol = jnp.sum(jnp.where(ecol == e, w_all, 0.0), axis=1, keepdims=True)
    out_ref[...] += o * wcol


def kernel(x, scale, gate_w, gate_b, mlp1_weight, mlp1_bias, mlp2_weight, mlp2_bias):
    B, D = x.shape
    E, twoF, _ = mlp1_weight.shape
    F = twoF // 2

    scale2 = scale.reshape(1, D)
    gate_b2 = gate_b.reshape(1, E)
    b1_3d = mlp1_bias.reshape(E, 1, twoF)
    b2_3d = mlp2_bias.reshape(E, 1, D)

    grid = (E,)
    out = pl.pallas_call(
        lambda *refs: _moe_body(F, *refs),
        grid=grid,
        in_specs=[
            pl.BlockSpec((B, D), lambda e: (0, 0)),
            pl.BlockSpec((1, D), lambda e: (0, 0)),
            pl.BlockSpec((E, D), lambda e: (0, 0)),
            pl.BlockSpec((1, E), lambda e: (0, 0)),
            pl.BlockSpec((1, twoF, D), lambda e: (e, 0, 0)),
            pl.BlockSpec((1, 1, twoF), lambda e: (e, 0, 0)),
            pl.BlockSpec((1, D, F), lambda e: (e, 0, 0)),
            pl.BlockSpec((1, 1, D), lambda e: (e, 0, 0)),
        ],
        out_specs=pl.BlockSpec((B, D), lambda e: (0, 0)),
        out_shape=jax.ShapeDtypeStruct((B, D), jnp.float32),
        scratch_shapes=[
            pltpu.VMEM((B, D), jnp.float32),
            pltpu.VMEM((B, E), jnp.float32),
        ],
        compiler_params=pltpu.CompilerParams(
            dimension_semantics=("arbitrary",),
        ),
    )(x, scale2, gate_w, gate_b2, mlp1_weight, b1_3d, mlp2_weight, b2_3d)
    return out


# speedup vs baseline: 1.0206x; 1.0206x over previous
import jax
import jax.numpy as jnp
from jax.experimental import pallas as pl
from jax.experimental.pallas import tpu as pltpu


def _routing_body(x_ref, scale_ref, gw_ref, gb_ref, t_ref, w_ref, meta_ref):
    x = x_ref[...]
    t = x * jax.lax.rsqrt(jnp.mean(x * x, axis=-1, keepdims=True) + 1e-5)
    t = t * scale_ref[...]
    t_ref[...] = t
    g = jax.lax.dot_general(
        t, gw_ref[...], (((1,), (1,)), ((), ())),
        preferred_element_type=jnp.float32) + gb_ref[...]

    E = g.shape[1]
    col = jax.lax.broadcasted_iota(jnp.int32, g.shape, 1)
    v1 = jnp.max(g, axis=-1, keepdims=True)
    e1 = jnp.min(jnp.where(g == v1, col, E), axis=-1, keepdims=True)
    first1 = (col == e1)
    g2 = jnp.where(first1, -1e30, g)
    v2 = jnp.max(g2, axis=-1, keepdims=True)
    e2 = jnp.min(jnp.where(g2 == v2, col, E), axis=-1, keepdims=True)
    first2 = (col == e2)
    p1 = jax.nn.sigmoid(v1 - v2)
    wmat = jnp.where(first1, p1, 0.0) + jnp.where(first2, 1.0 - p1, 0.0)
    w_ref[...] = wmat

    act = (jnp.max(wmat, axis=0, keepdims=True) > 0.0)
    r2 = jax.lax.broadcasted_iota(jnp.int32, (E, E), 0)
    c2 = jax.lax.broadcasted_iota(jnp.int32, (E, E), 1)
    lower_tri = (r2 <= c2).astype(jnp.float32)
    pos = jax.lax.dot_general(
        act.astype(jnp.float32), lower_tri, (((1,), (0,)), ((), ())),
        preferred_element_type=jnp.float32)
    na = jnp.max(pos, axis=1, keepdims=True)
    colv = jax.lax.broadcasted_iota(jnp.int32, (1, E), 1).astype(jnp.float32)
    elast = jnp.sum(jnp.where(act & (pos == na), colv, 0.0),
                    axis=1, keepdims=True)
    rowi = jax.lax.broadcasted_iota(jnp.int32, (E + 1, E), 0).astype(jnp.float32)
    match = (jnp.broadcast_to(pos, (E + 1, E)) == rowi + 1.0) \
        & jnp.broadcast_to(act, (E + 1, E))
    cole = jax.lax.broadcasted_iota(jnp.int32, (E + 1, E), 1).astype(jnp.float32)
    vals = jnp.sum(jnp.where(match, cole, 0.0), axis=1, keepdims=True)
    rows1 = jax.lax.broadcasted_iota(jnp.int32, (E + 1, 1), 0).astype(jnp.float32)
    meta = jnp.where(rows1 == float(E), na,
                     vals + jnp.where(rows1 >= na, elast, 0.0))
    meta_ref[...] = meta.astype(jnp.int32)


def _expert_body(E, F, meta_ref, x_ref, t_ref, W_ref,
                 w1_ref, b1_ref, w2_ref, b2_ref, out_ref):
    i = pl.program_id(0)
    na = meta_ref[E]
    e = meta_ref[i]

    @pl.when(i == 0)
    def _init():
        out_ref[...] = x_ref[...]

    @pl.when(i < na)
    def _accum():
        t = t_ref[...]
        h = jax.lax.dot_general(
            t.astype(jnp.bfloat16), w1_ref[0].astype(jnp.bfloat16),
            (((1,), (1,)), ((), ())),
            preferred_element_type=jnp.float32) + b1_ref[0]
        x_glu = h[:, :F]
        x_lin = h[:, F:]
        a = x_glu * jax.nn.sigmoid(1.702 * x_glu) * (x_lin + 1.0)
        o = jax.lax.dot_general(
            a.astype(jnp.bfloat16), w2_ref[0].astype(jnp.bfloat16),
            (((1,), (1,)), ((), ())),
            preferred_element_type=jnp.float32) + b2_ref[0]
        w_all = W_ref[...]
        ecol = jax.lax.broadcasted_iota(jnp.int32, w_all.shape, 1)
        wcol = jnp.sum(jnp.where(ecol == e, w_all, 0.0), axis=1, keepdims=True)
        out_ref[...] += o * wcol


def kernel(x, scale, gate_w, gate_b, mlp1_weight, mlp1_bias, mlp2_weight, mlp2_bias):
    B, D = x.shape
    E, twoF, _ = mlp1_weight.shape
    F = twoF // 2

    scale2 = scale.reshape(1, D)
    gate_b2 = gate_b.reshape(1, E)
    b1_3d = mlp1_bias.reshape(E, 1, twoF)
    b2_3d = mlp2_bias.reshape(E, 1, D)

    t, W, meta2d = pl.pallas_call(
        _routing_body,
        in_specs=[
            pl.BlockSpec((B, D), lambda: (0, 0)),
            pl.BlockSpec((1, D), lambda: (0, 0)),
            pl.BlockSpec((E, D), lambda: (0, 0)),
            pl.BlockSpec((1, E), lambda: (0, 0)),
        ],
        out_specs=[
            pl.BlockSpec((B, D), lambda: (0, 0)),
            pl.BlockSpec((B, E), lambda: (0, 0)),
            pl.BlockSpec((E + 1, 1), lambda: (0, 0)),
        ],
        out_shape=[
            jax.ShapeDtypeStruct((B, D), jnp.float32),
            jax.ShapeDtypeStruct((B, E), jnp.float32),
            jax.ShapeDtypeStruct((E + 1, 1), jnp.int32),
        ],
    )(x, scale2, gate_w, gate_b2)
    meta = meta2d.reshape(E + 1)

    grid_spec = pltpu.PrefetchScalarGridSpec(
        num_scalar_prefetch=1,
        grid=(E,),
        in_specs=[
            pl.BlockSpec((B, D), lambda i, m: (0, 0)),
            pl.BlockSpec((B, D), lambda i, m: (0, 0)),
            pl.BlockSpec((B, E), lambda i, m: (0, 0)),
            pl.BlockSpec((1, twoF, D), lambda i, m: (m[i], 0, 0)),
            pl.BlockSpec((1, 1, twoF), lambda i, m: (m[i], 0, 0)),
            pl.BlockSpec((1, D, F), lambda i, m: (m[i], 0, 0)),
            pl.BlockSpec((1, 1, D), lambda i, m: (m[i], 0, 0)),
        ],
        out_specs=pl.BlockSpec((B, D), lambda i, m: (0, 0)),
    )
    out = pl.pallas_call(
        lambda *refs: _expert_body(E, F, *refs),
        grid_spec=grid_spec,
        out_shape=jax.ShapeDtypeStruct((B, D), jnp.float32),
        compiler_params=pltpu.CompilerParams(
            dimension_semantics=("arbitrary",),
        ),
    )(meta, x, t, W, mlp1_weight, b1_3d, mlp2_weight, b2_3d)
    return out


# speedup vs baseline: 1.1120x; 1.0895x over previous
import jax
import jax.numpy as jnp
from jax.experimental import pallas as pl
from jax.experimental.pallas import tpu as pltpu


def _body(E, F, B, D,
          x_ref, scale_ref, gw_ref, gb_ref, w1_hbm, b1_hbm, w2_hbm, b2_hbm,
          out_ref,
          t_scr, w_scr, meta_vmem, meta_smem,
          w1buf, b1buf, w2buf, b2buf, meta_sem, copy_sems):
    i = pl.program_id(0)
    twoF = 2 * F

    @pl.when(i == 0)
    def _routing():
        x = x_ref[...]
        t = x * jax.lax.rsqrt(jnp.mean(x * x, axis=-1, keepdims=True) + 1e-5)
        t = t * scale_ref[...]
        t_scr[...] = t
        g = jax.lax.dot_general(
            t, gw_ref[...], (((1,), (1,)), ((), ())),
            preferred_element_type=jnp.float32) + gb_ref[...]

        col = jax.lax.broadcasted_iota(jnp.int32, g.shape, 1)
        v1 = jnp.max(g, axis=-1, keepdims=True)
        e1 = jnp.min(jnp.where(g == v1, col, E), axis=-1, keepdims=True)
        first1 = (col == e1)
        g2 = jnp.where(first1, -1e30, g)
        v2 = jnp.max(g2, axis=-1, keepdims=True)
        e2 = jnp.min(jnp.where(g2 == v2, col, E), axis=-1, keepdims=True)
        first2 = (col == e2)
        p1 = jax.nn.sigmoid(v1 - v2)
        wmat = jnp.where(first1, p1, 0.0) + jnp.where(first2, 1.0 - p1, 0.0)
        w_scr[...] = wmat

        act = (jnp.max(wmat, axis=0, keepdims=True) > 0.0)
        r2 = jax.lax.broadcasted_iota(jnp.int32, (E, E), 0)
        c2 = jax.lax.broadcasted_iota(jnp.int32, (E, E), 1)
        lower_tri = (r2 <= c2).astype(jnp.float32)
        pos = jax.lax.dot_general(
            act.astype(jnp.float32), lower_tri, (((1,), (0,)), ((), ())),
            preferred_element_type=jnp.float32)
        na = jnp.max(pos, axis=1, keepdims=True)
        rowi = jax.lax.broadcasted_iota(jnp.int32, (E + 1, E), 0).astype(jnp.float32)
        match = (jnp.broadcast_to(pos, (E + 1, E)) == rowi + 1.0) \
            & jnp.broadcast_to(act, (E + 1, E))
        cole = jax.lax.broadcasted_iota(jnp.int32, (E + 1, E), 1).astype(jnp.float32)
        vals = jnp.sum(jnp.where(match, cole, 0.0), axis=1, keepdims=True)
        rows1 = jax.lax.broadcasted_iota(jnp.int32, (E + 1, 1), 0).astype(jnp.float32)
        meta = jnp.where(rows1 == float(E), na, vals)
        meta_vmem[...] = meta.astype(jnp.int32)
        pltpu.make_async_copy(meta_vmem, meta_smem, meta_sem).start()
        pltpu.make_async_copy(meta_vmem, meta_smem, meta_sem).wait()

        e0 = meta_smem[0, 0]
        pltpu.make_async_copy(w1_hbm.at[e0], w1buf.at[0], copy_sems.at[0, 0]).start()
        pltpu.make_async_copy(w2_hbm.at[e0], w2buf.at[0], copy_sems.at[0, 1]).start()
        pltpu.make_async_copy(b1_hbm.at[e0], b1buf.at[0], copy_sems.at[0, 2]).start()
        pltpu.make_async_copy(b2_hbm.at[e0], b2buf.at[0], copy_sems.at[0, 3]).start()
        out_ref[...] = x

    na = meta_smem[E, 0]

    @pl.when(i + 1 < na)
    def _prefetch():
        en = meta_smem[i + 1, 0]
        s = (i + 1) % 2
        pltpu.make_async_copy(w1_hbm.at[en], w1buf.at[s], copy_sems.at[s, 0]).start()
        pltpu.make_async_copy(w2_hbm.at[en], w2buf.at[s], copy_sems.at[s, 1]).start()
        pltpu.make_async_copy(b1_hbm.at[en], b1buf.at[s], copy_sems.at[s, 2]).start()
        pltpu.make_async_copy(b2_hbm.at[en], b2buf.at[s], copy_sems.at[s, 3]).start()

    @pl.when(i < na)
    def _accum():
        e = meta_smem[i, 0]
        s = i % 2
        pltpu.make_async_copy(w1_hbm.at[e], w1buf.at[s], copy_sems.at[s, 0]).wait()
        pltpu.make_async_copy(w2_hbm.at[e], w2buf.at[s], copy_sems.at[s, 1]).wait()
        pltpu.make_async_copy(b1_hbm.at[e], b1buf.at[s], copy_sems.at[s, 2]).wait()
        pltpu.make_async_copy(b2_hbm.at[e], b2buf.at[s], copy_sems.at[s, 3]).wait()
        t = t_scr[...]
        h = jax.lax.dot_general(
            t.astype(jnp.bfloat16), w1buf[s].astype(jnp.bfloat16),
            (((1,), (1,)), ((), ())),
            preferred_element_type=jnp.float32) + b1buf[s]
        x_glu = h[:, :F]
        x_lin = h[:, F:]
        a = x_glu * jax.nn.sigmoid(1.702 * x_glu) * (x_lin + 1.0)
        o = jax.lax.dot_general(
            a.astype(jnp.bfloat16), w2buf[s].astype(jnp.bfloat16),
            (((1,), (1,)), ((), ())),
            preferred_element_type=jnp.float32) + b2buf[s]
        w_all = w_scr[...]
        ecol = jax.lax.broadcasted_iota(jnp.int32, w_all.shape, 1)
        wcol = jnp.sum(jnp.where(ecol == e, w_all, 0.0), axis=1, keepdims=True)
        out_ref[...] += o * wcol


def kernel(x, scale, gate_w, gate_b, mlp1_weight, mlp1_bias, mlp2_weight, mlp2_bias):
    B, D = x.shape
    E, twoF, _ = mlp1_weight.shape
    F = twoF // 2

    scale2 = scale.reshape(1, D)
    gate_b2 = gate_b.reshape(1, E)
    b1_3d = mlp1_bias.reshape(E, 1, twoF)
    b2_3d = mlp2_bias.reshape(E, 1, D)

    out = pl.pallas_call(
        lambda *refs: _body(E, F, B, D, *refs),
        grid=(E,),
        in_specs=[
            pl.BlockSpec((B, D), lambda i: (0, 0)),
            pl.BlockSpec((1, D), lambda i: (0, 0)),
            pl.BlockSpec((E, D), lambda i: (0, 0)),
            pl.BlockSpec((1, E), lambda i: (0, 0)),
            pl.BlockSpec(memory_space=pltpu.MemorySpace.HBM),
            pl.BlockSpec(memory_space=pltpu.MemorySpace.HBM),
            pl.BlockSpec(memory_space=pltpu.MemorySpace.HBM),
            pl.BlockSpec(memory_space=pltpu.MemorySpace.HBM),
        ],
        out_specs=pl.BlockSpec((B, D), lambda i: (0, 0)),
        out_shape=jax.ShapeDtypeStruct((B, D), jnp.float32),
        scratch_shapes=[
            pltpu.VMEM((B, D), jnp.float32),
            pltpu.VMEM((B, E), jnp.float32),
            pltpu.VMEM((E + 1, 1), jnp.int32),
            pltpu.SMEM((E + 1, 1), jnp.int32),
            pltpu.VMEM((2, twoF, D), jnp.float32),
            pltpu.VMEM((2, 1, twoF), jnp.float32),
            pltpu.VMEM((2, D, F), jnp.float32),
            pltpu.VMEM((2, 1, D), jnp.float32),
            pltpu.SemaphoreType.DMA,
            pltpu.SemaphoreType.DMA((2, 4)),
        ],
        compiler_params=pltpu.CompilerParams(
            dimension_semantics=("arbitrary",),
        ),
    )(x, scale2, gate_w, gate_b2, mlp1_weight, b1_3d, mlp2_weight, b2_3d)
    return out


# speedup vs baseline: 1.1580x; 1.0414x over previous
import jax
import jax.numpy as jnp
from jax.experimental import pallas as pl
from jax.experimental.pallas import tpu as pltpu


def _start_expert_copies(e, s, w1_hbm, b1_hbm, w2_hbm, b2_hbm,
                         w1buf, b1buf, w2buf, b2buf, copy_sems):
    pltpu.make_async_copy(w1_hbm.at[e], w1buf.at[s], copy_sems.at[s, 0]).start()
    pltpu.make_async_copy(w2_hbm.at[e], w2buf.at[s], copy_sems.at[s, 1]).start()
    pltpu.make_async_copy(b1_hbm.at[e], b1buf.at[s], copy_sems.at[s, 2]).start()
    pltpu.make_async_copy(b2_hbm.at[e], b2buf.at[s], copy_sems.at[s, 3]).start()


def _wait_expert_copies(e, s, w1_hbm, b1_hbm, w2_hbm, b2_hbm,
                        w1buf, b1buf, w2buf, b2buf, copy_sems):
    pltpu.make_async_copy(w1_hbm.at[e], w1buf.at[s], copy_sems.at[s, 0]).wait()
    pltpu.make_async_copy(w2_hbm.at[e], w2buf.at[s], copy_sems.at[s, 1]).wait()
    pltpu.make_async_copy(b1_hbm.at[e], b1buf.at[s], copy_sems.at[s, 2]).wait()
    pltpu.make_async_copy(b2_hbm.at[e], b2buf.at[s], copy_sems.at[s, 3]).wait()


def _body(E, F, B, D,
          x_ref, scale_ref, gw_ref, gb_ref, w1_hbm, b1_hbm, w2_hbm, b2_hbm,
          out_ref,
          t_scr, w_scr, meta_vmem, meta_smem,
          w1buf, b1buf, w2buf, b2buf, meta_sem, copy_sems):
    bufs = (w1_hbm, b1_hbm, w2_hbm, b2_hbm, w1buf, b1buf, w2buf, b2buf,
            copy_sems)

    x = x_ref[...]
    t = x * jax.lax.rsqrt(jnp.mean(x * x, axis=-1, keepdims=True) + 1e-5)
    t = t * scale_ref[...]
    t_scr[...] = t
    g = jax.lax.dot_general(
        t, gw_ref[...], (((1,), (1,)), ((), ())),
        preferred_element_type=jnp.float32) + gb_ref[...]

    col = jax.lax.broadcasted_iota(jnp.int32, g.shape, 1)
    v1 = jnp.max(g, axis=-1, keepdims=True)
    e1 = jnp.min(jnp.where(g == v1, col, E), axis=-1, keepdims=True)
    first1 = (col == e1)
    g2 = jnp.where(first1, -1e30, g)
    v2 = jnp.max(g2, axis=-1, keepdims=True)
    e2 = jnp.min(jnp.where(g2 == v2, col, E), axis=-1, keepdims=True)
    first2 = (col == e2)
    p1 = jax.nn.sigmoid(v1 - v2)
    wmat = jnp.where(first1, p1, 0.0) + jnp.where(first2, 1.0 - p1, 0.0)
    w_scr[...] = wmat

    act = (jnp.max(wmat, axis=0, keepdims=True) > 0.0)
    r2 = jax.lax.broadcasted_iota(jnp.int32, (E, E), 0)
    c2 = jax.lax.broadcasted_iota(jnp.int32, (E, E), 1)
    lower_tri = (r2 <= c2).astype(jnp.float32)
    pos = jax.lax.dot_general(
        act.astype(jnp.float32), lower_tri, (((1,), (0,)), ((), ())),
        preferred_element_type=jnp.float32)
    na_f = jnp.max(pos, axis=1, keepdims=True)
    rowi = jax.lax.broadcasted_iota(jnp.int32, (E + 1, E), 0).astype(jnp.float32)
    match = (jnp.broadcast_to(pos, (E + 1, E)) == rowi + 1.0) \
        & jnp.broadcast_to(act, (E + 1, E))
    cole = jax.lax.broadcasted_iota(jnp.int32, (E + 1, E), 1).astype(jnp.float32)
    vals = jnp.sum(jnp.where(match, cole, 0.0), axis=1, keepdims=True)
    rows1 = jax.lax.broadcasted_iota(jnp.int32, (E + 1, 1), 0).astype(jnp.float32)
    meta = jnp.where(rows1 == float(E), na_f, vals)
    meta_vmem[...] = meta.astype(jnp.int32)
    pltpu.make_async_copy(meta_vmem, meta_smem, meta_sem).start()
    pltpu.make_async_copy(meta_vmem, meta_smem, meta_sem).wait()

    na = meta_smem[E, 0]
    out_ref[...] = x

    _start_expert_copies(meta_smem[0, 0], 0, *bufs)

    @pl.when(na > 1)
    def _pf1():
        _start_expert_copies(meta_smem[1, 0], 1, *bufs)

    def loop_body(s, carry):
        e = meta_smem[s, 0]
        sl = jax.lax.rem(s, 2)
        _wait_expert_copies(e, sl, *bufs)
        h = jax.lax.dot_general(
            t_scr[...].astype(jnp.bfloat16), w1buf[sl].astype(jnp.bfloat16),
            (((1,), (1,)), ((), ())),
            preferred_element_type=jnp.float32) + b1buf[sl]
        x_glu = h[:, :F]
        x_lin = h[:, F:]
        a = x_glu * jax.nn.sigmoid(1.702 * x_glu) * (x_lin + 1.0)
        o = jax.lax.dot_general(
            a.astype(jnp.bfloat16), w2buf[sl].astype(jnp.bfloat16),
            (((1,), (1,)), ((), ())),
            preferred_element_type=jnp.float32) + b2buf[sl]
        w_all = w_scr[...]
        ecol = jax.lax.broadcasted_iota(jnp.int32, w_all.shape, 1)
        wcol = jnp.sum(jnp.where(ecol == e, w_all, 0.0), axis=1, keepdims=True)
        out_ref[...] += o * wcol

        @pl.when(s + 2 < na)
        def _pf():
            _start_expert_copies(meta_smem[s + 2, 0], sl, *bufs)

        return carry

    jax.lax.fori_loop(0, na, loop_body, 0)


def kernel(x, scale, gate_w, gate_b, mlp1_weight, mlp1_bias, mlp2_weight, mlp2_bias):
    B, D = x.shape
    E, twoF, _ = mlp1_weight.shape
    F = twoF // 2

    scale2 = scale.reshape(1, D)
    gate_b2 = gate_b.reshape(1, E)
    b1_3d = mlp1_bias.reshape(E, 1, twoF)
    b2_3d = mlp2_bias.reshape(E, 1, D)

    out = pl.pallas_call(
        lambda *refs: _body(E, F, B, D, *refs),
        in_specs=[
            pl.BlockSpec((B, D), lambda: (0, 0)),
            pl.BlockSpec((1, D), lambda: (0, 0)),
            pl.BlockSpec((E, D), lambda: (0, 0)),
            pl.BlockSpec((1, E), lambda: (0, 0)),
            pl.BlockSpec(memory_space=pltpu.MemorySpace.HBM),
            pl.BlockSpec(memory_space=pltpu.MemorySpace.HBM),
            pl.BlockSpec(memory_space=pltpu.MemorySpace.HBM),
            pl.BlockSpec(memory_space=pltpu.MemorySpace.HBM),
        ],
        out_specs=pl.BlockSpec((B, D), lambda: (0, 0)),
        out_shape=jax.ShapeDtypeStruct((B, D), jnp.float32),
        scratch_shapes=[
            pltpu.VMEM((B, D), jnp.float32),
            pltpu.VMEM((B, E), jnp.float32),
            pltpu.VMEM((E + 1, 1), jnp.int32),
            pltpu.SMEM((E + 1, 1), jnp.int32),
            pltpu.VMEM((2, twoF, D), jnp.float32),
            pltpu.VMEM((2, 1, twoF), jnp.float32),
            pltpu.VMEM((2, D, F), jnp.float32),
            pltpu.VMEM((2, 1, D), jnp.float32),
            pltpu.SemaphoreType.DMA,
            pltpu.SemaphoreType.DMA((2, 4)),
        ],
    )(x, scale2, gate_w, gate_b2, mlp1_weight, b1_3d, mlp2_weight, b2_3d)
    return out
